# Initial kernel scaffold; baseline (speedup 1.0000x reference)
import jax
import jax.numpy as jnp
from jax.experimental import pallas as pl


def kernel(input, index, value):
    raise NotImplementedError("write your pallas kernel here")


# speedup vs baseline: 1.3202x; 1.3202x over previous
import jax
import jax.numpy as jnp
from jax import lax
from jax.experimental import pallas as pl
from jax.experimental.pallas import tpu as pltpu
from jax.experimental.pallas import tpu_sc as plsc

N_ROWS = 50000
N_UPD = 16384
ROW_W = 1024
NC, NS = 2, 16
NW = NC * NS
RPW = 1568
CP = 32
NVEC = N_UPD // 16
LIST_CAP = 1664


def _sc_body(inp_hbm, idx_hbm, val_hbm, out_hbm,
             idxv, winner, cbuf, vbuf, rlist, wlist, sem0, sem1):
    i32 = jnp.int32
    c16 = i32(16)
    wid = (lax.axis_index("s").astype(i32) * i32(NC)
           + lax.axis_index("c").astype(i32))
    start = wid * i32(RPW)
    end = jnp.minimum(start + i32(RPW), i32(N_ROWS))
    size = end - start
    lane = lax.iota(i32, 16)

    ncp = (size + i32(CP - 1)) // i32(CP)

    def copy_body(c, carry):
        cs = jnp.minimum(start + c * i32(CP), end - i32(CP))
        pltpu.sync_copy(inp_hbm.at[pl.ds(cs, CP)], cbuf)
        pltpu.sync_copy(cbuf, out_hbm.at[pl.ds(cs, CP)])
        return carry

    lax.fori_loop(i32(0), ncp, copy_body, i32(0))

    pltpu.sync_copy(idx_hbm, idxv)
    neg1 = jnp.full((16,), -1, i32)
    ninit = (size + i32(15)) // c16

    def init_body(v, carry):
        winner[pl.ds(start + v * c16, 16)] = neg1
        return carry

    lax.fori_loop(i32(0), ninit, init_body, i32(0))

    def win_body(t, carry):
        v = idxv[pl.ds(t * c16, 16)]
        ivec = lane + t * c16
        plsc.store_scatter(winner, [v], ivec)
        rb = plsc.load_gather(winner, [v])
        anyb = jnp.max(jnp.where(rb != ivec, i32(1), i32(0)))

        @pl.when(anyb > 0)
        def _fix():
            for l in range(16):
                plsc.store_scatter(winner, [v], ivec, mask=lane == l)

        return carry

    lax.fori_loop(i32(0), i32(NVEC), win_body, i32(0))

    def comp_body(v, off):
        base = start + v * c16
        w = winner[pl.ds(base, 16)]
        rvec = lane + base
        m = (w >= 0) & (rvec < end)
        mi = m.astype(i32)
        pos = off + lax.cumsum(mi) - 1
        plsc.store_scatter(rlist, [pos], rvec, mask=m)
        plsc.store_scatter(wlist, [pos], w, mask=m)
        return off + jnp.sum(mi, dtype=i32)

    off = lax.fori_loop(i32(0), ninit, comp_body, i32(0))

    rem = lax.rem(off, c16)

    @pl.when(rem > 0)
    def _pad():
        lastpos = jnp.full((16,), off - 1, i32)
        lastr = plsc.load_gather(rlist, [lastpos])
        lastw = plsc.load_gather(wlist, [lastpos])
        padpos = off + lane
        padmask = lane < (c16 - rem)
        plsc.store_scatter(rlist, [padpos], lastr, mask=padmask)
        plsc.store_scatter(wlist, [padpos], lastw, mask=padmask)

    nchunks = (off + i32(15)) // c16

    def sc_body(c, carry):
        rv = rlist[pl.ds(c * c16, 16)]
        wv = wlist[pl.ds(c * c16, 16)]
        pltpu.async_copy(val_hbm.at[wv], vbuf, sem0).wait()
        pltpu.async_copy(vbuf, out_hbm.at[rv], sem1).wait()
        return carry

    lax.fori_loop(i32(0), nchunks, sc_body, i32(0))


def kernel(input, index, value):
    inp32 = lax.bitcast_convert_type(input, jnp.int32).reshape(N_ROWS, ROW_W)
    val32 = lax.bitcast_convert_type(value, jnp.int32).reshape(N_UPD, ROW_W)
    idx32 = index.astype(jnp.int32)

    mesh = plsc.VectorSubcoreMesh(core_axis_name="c", subcore_axis_name="s")
    scatter = pl.kernel(
        _sc_body,
        out_type=jax.ShapeDtypeStruct((N_ROWS, ROW_W), jnp.int32),
        mesh=mesh,
        compiler_params=pltpu.CompilerParams(needs_layout_passes=False),
        scratch_types=[
            pltpu.VMEM((N_UPD,), jnp.int32),
            pltpu.VMEM((N_ROWS + 48,), jnp.int32),
            pltpu.VMEM((CP, ROW_W), jnp.int32),
            pltpu.VMEM((16, ROW_W), jnp.int32),
            pltpu.VMEM((LIST_CAP,), jnp.int32),
            pltpu.VMEM((LIST_CAP,), jnp.int32),
            pltpu.SemaphoreType.DMA,
            pltpu.SemaphoreType.DMA,
        ],
    )
    out32 = scatter(inp32, idx32, val32)
    return lax.bitcast_convert_type(
        out32.reshape(N_ROWS, 64, 8, 2), jnp.int64)
